# baseline (speedup 1.0000x reference)
import functools

import jax
import jax.numpy as jnp
from jax import lax
from jax.experimental import pallas as pl
from jax.experimental.pallas import tpu as pltpu
from jax.experimental.pallas import tpu_sc as plsc

_B, _L = 16384, 200

_INFO = plsc.get_sparse_core_info()
_NC = _INFO.num_cores
_NS = _INFO.num_subcores
_LANES = _INFO.num_lanes
_NW = _NC * _NS
_COLS_W = _B // _NW
assert _COLS_W * _NW == _B and _COLS_W % _LANES == 0


def _make_masks_equal_kernel():
    mesh = plsc.VectorSubcoreMesh(core_axis_name="c", subcore_axis_name="s")

    @functools.partial(
        pl.kernel,
        mesh=mesh,
        out_type=jax.ShapeDtypeStruct((_NW, _LANES), jnp.int32),
        scratch_types=[
            pltpu.VMEM((_L, _COLS_W), jnp.int32),
            pltpu.VMEM((_LANES,), jnp.int32),
            pltpu.SemaphoreType.DMA,
        ],
        compiler_params=pltpu.CompilerParams(use_tc_tiling_on_sc=True),
    )
    def masks_equal_kernel(tokens_hbm, out_hbm, buf, res, sem):
        wid = lax.axis_index("s") * _NC + lax.axis_index("c")
        base = wid * _COLS_W
        pltpu.async_copy(
            tokens_hbm.at[:, pl.ds(base, _COLS_W)], buf, sem
        ).wait()

        def step(r, acc):
            for v in range(_COLS_W // _LANES):
                x = buf[r, pl.ds(v * _LANES, _LANES)]
                input_mask = x != 0
                random_mask = jnp.zeros((_LANES,), jnp.bool_)
                mask_no_alter = jnp.logical_and(random_mask, input_mask)
                mask_alter = jnp.logical_and(random_mask, input_mask)
                eq = mask_no_alter == mask_alter
                acc = jnp.logical_and(acc, eq)
            return acc

        acc = lax.fori_loop(0, _L, step, jnp.ones((_LANES,), jnp.bool_))
        res[...] = acc.astype(jnp.int32)
        pltpu.sync_copy(res, out_hbm.at[wid])

    return masks_equal_kernel


_MASKS_EQUAL = _make_masks_equal_kernel()


def kernel(inputs, table):
    del table
    partial = _MASKS_EQUAL(inputs.T)
    return jnp.all(partial == 1)
